# baseline (speedup 1.0000x reference)
import jax
import jax.numpy as jnp
from jax.experimental import pallas as pl
from jax.experimental.pallas import tpu as pltpu

INF = 1000000000000.0
B, H, EMB = 64, 512, 512
V = 32000
S_ENC, S_SEQ = 512, 512
ENC2 = 512
NUM_OOV = 50
A_DIM, R_DIM = 128, 256

SC_ENC = 32
NC_ENC = S_ENC // SC_ENC
SC_SEQ = 32
NC_SEQ = S_SEQ // SC_SEQ
BC_SC = 8
NB_SC = B // BC_SC
VC = 1280
NVC = V // VC

HI = jax.lax.Precision.HIGHEST


def _mmT(x, w, precision=None):
    return jax.lax.dot_general(
        x, w, (((1,), (1,)), ((), ())),
        precision=precision, preferred_element_type=jnp.float32)


def _gather_body(ids_ref, emb_ref, out_ref):
    del ids_ref
    out_ref[...] = emb_ref[...]


def _lstm_body(emb_ref, pc_ref, h0_ref, c0_ref, wr_ref, wih_ref, whh_ref,
               br_ref, bih_ref, bhh_ref, wkey_ref, bkey_ref,
               h1_ref, c1_ref, q_ref, qb_ref):
    xin = jnp.concatenate([emb_ref[...], pc_ref[...]], axis=1)
    x = _mmT(xin, wr_ref[...], HI) + br_ref[...]
    gates = (_mmT(x, wih_ref[...], HI) + _mmT(h0_ref[...], whh_ref[...], HI)
             + bih_ref[...] + bhh_ref[...])
    i = jax.nn.sigmoid(gates[:, 0:H])
    f = jax.nn.sigmoid(gates[:, H:2 * H])
    g = jnp.tanh(gates[:, 2 * H:3 * H])
    o = jax.nn.sigmoid(gates[:, 3 * H:4 * H])
    c1 = f * c0_ref[...] + i * g
    h1 = o * jnp.tanh(c1)
    h1_ref[...] = h1
    c1_ref[...] = c1
    q_ref[...] = jnp.dot(h1, wkey_ref[...], precision=HI,
                         preferred_element_type=jnp.float32)
    qb_ref[...] = jnp.sum(h1 * bkey_ref[...], axis=1, keepdims=True)


def _encatt_body(enc_ref, q_ref, qb_ref, e_ref, m_ref, s_ref, ctx_ref):
    enc = enc_ref[...]
    mask = jnp.sum(enc, axis=2) == 0.0
    e = jnp.sum(enc * q_ref[...][:, None, :], axis=2) + qb_ref[...]
    e = jnp.where(mask, 1e-12, e)
    e_ref[0] = e
    m = jnp.max(e, axis=1)
    p = jnp.exp(e - m[:, None])
    m_ref[0, 0, :] = m
    s_ref[0, 0, :] = jnp.sum(p, axis=1)
    ctx_ref[0] = jnp.sum(p[:, :, None] * enc, axis=1)


def _combine_body(e_ref, m_ref, s_ref, cp_ref, h1_ref, wc_ref, bc_ref,
                  attn_ref, ctx_ref, comb_ref):
    m = m_ref[...][:, 0, :]
    M = jnp.max(m, axis=0)
    scale = jnp.exp(m - M[None, :])
    denom = jnp.sum(s_ref[...][:, 0, :] * scale, axis=0)
    attn_ref[...] = jnp.exp(e_ref[...] - M[None, :, None]) / denom[None, :, None]
    ctx = jnp.sum(cp_ref[...] * scale[:, :, None], axis=0) / denom[:, None]
    ctx_ref[...] = ctx
    hin = jnp.concatenate([h1_ref[...], ctx], axis=1)
    comb_ref[...] = jnp.tanh(_mmT(hin, wc_ref[...], HI) + bc_ref[...])


def _seqe_body(seq_ref, wsk_ref, h1_ref, mask_ref, e_ref):
    x = seq_ref[...].reshape(B * SC_SEQ, ENC2)
    sk = jnp.tanh(_mmT(x, wsk_ref[...]))
    sk = sk.reshape(B, SC_SEQ, H)
    e = jnp.sum(sk * h1_ref[...][:, None, :], axis=2)
    e_ref[0] = jnp.where(mask_ref[0] != 0.0, 1e-12, e)


def _scatter_body(e_ref, idx_ref, out_ref):
    e = e_ref[...]
    idx = idx_ref[...]
    eq = idx[:, :, None] == idx[:, None, :]
    gmax = jnp.max(jnp.where(eq, e[:, None, :], -INF), axis=2)
    sidx = jax.lax.broadcasted_iota(jnp.int32, (BC_SC, S_SEQ, S_SEQ), 2)
    cand = jnp.where(eq & (e[:, None, :] == gmax[:, :, None]), sidx, S_SEQ)
    first = jnp.min(cand, axis=2)
    winner = first == jax.lax.broadcasted_iota(jnp.int32, (BC_SC, S_SEQ), 1)
    val = jnp.where(winner, e, 0.0)
    a = idx // R_DIM
    r = idx % R_DIM
    a_oh = (a[:, :, None] ==
            jax.lax.broadcasted_iota(jnp.int32, (BC_SC, S_SEQ, A_DIM), 2)
            ).astype(jnp.float32)
    r_oh = (r[:, :, None] ==
            jax.lax.broadcasted_iota(jnp.int32, (BC_SC, S_SEQ, R_DIM), 2)
            ).astype(jnp.float32)
    av = a_oh * val[:, :, None]
    out_ref[...] = jax.lax.dot_general(
        av, r_oh, (((1,), (1,)), ((0,), (0,))),
        precision=HI, preferred_element_type=jnp.float32)


def _logits_body(comb_ref, w_ref, b_ref, cp_ref, o_ref):
    lg = _mmT(comb_ref[...], w_ref[...])
    o_ref[...] = lg + b_ref[...] + cp_ref[...]


def _lsm_body(main_ref, oov_ref, o_ref):
    x = main_ref[...]
    x = jnp.where(x == 0.0, -INF, x)
    y = oov_ref[...]
    y = jnp.where(y == 0.0, -INF, y)
    m = jnp.maximum(jnp.max(x, axis=1), jnp.max(y, axis=1))
    lse = jnp.log(jnp.sum(jnp.exp(x - m[:, None]), axis=1)
                  + jnp.sum(jnp.exp(y - m[:, None]), axis=1))
    sub = (m + lse)[:, None]
    o_ref[:, 0:V] = x - sub
    o_ref[:, V:V + NUM_OOV] = y - sub


def kernel(input_ids, prev_context, h0, c0, encoder_outputs, seqs_encoding,
           seqs_encoding_mask, ext_idx, embed, W_reduce, b_reduce, W_key,
           b_key, W_comb, b_comb, W_seqkey, W_ih, W_hh, b_ih, b_hh, W_out,
           b_out):
    f32 = jnp.float32
    ids32 = input_ids.astype(jnp.int32)
    idx32 = ext_idx.astype(jnp.int32)
    maskf = seqs_encoding_mask.astype(f32)
    h0b, c0b = h0[0], c0[0]
    br = b_reduce.reshape(1, EMB)
    bih = b_ih.reshape(1, 4 * H)
    bhh = b_hh.reshape(1, 4 * H)
    bkey = b_key.reshape(1, H)
    bcomb = b_comb.reshape(1, H)
    bout = b_out.reshape(1, V)

    embedded = pl.pallas_call(
        _gather_body,
        grid_spec=pltpu.PrefetchScalarGridSpec(
            num_scalar_prefetch=1,
            grid=(B,),
            in_specs=[pl.BlockSpec((1, 1, EMB), lambda i, ids: (ids[i], 0, 0))],
            out_specs=pl.BlockSpec((1, 1, EMB), lambda i, ids: (i, 0, 0)),
        ),
        out_shape=jax.ShapeDtypeStruct((B, 1, EMB), f32),
        compiler_params=pltpu.CompilerParams(
            dimension_semantics=("arbitrary",)),
        name="embed_gather",
    )(ids32, embed.reshape(V, 1, EMB)).reshape(B, EMB)

    h1, c1, q, qb = pl.pallas_call(
        _lstm_body,
        out_shape=(
            jax.ShapeDtypeStruct((B, H), f32),
            jax.ShapeDtypeStruct((B, H), f32),
            jax.ShapeDtypeStruct((B, H), f32),
            jax.ShapeDtypeStruct((B, 1), f32),
        ),
        name="lstm_step",
    )(embedded, prev_context, h0b, c0b, W_reduce, W_ih, W_hh,
      br, bih, bhh, W_key, bkey)

    energies, m_parts, s_parts, ctx_parts = pl.pallas_call(
        _encatt_body,
        grid=(NC_ENC,),
        in_specs=[
            pl.BlockSpec((B, SC_ENC, H), lambda i: (0, i, 0)),
            pl.BlockSpec((B, H), lambda i: (0, 0)),
            pl.BlockSpec((B, 1), lambda i: (0, 0)),
        ],
        out_specs=(
            pl.BlockSpec((1, B, SC_ENC), lambda i: (i, 0, 0)),
            pl.BlockSpec((1, 1, B), lambda i: (i, 0, 0)),
            pl.BlockSpec((1, 1, B), lambda i: (i, 0, 0)),
            pl.BlockSpec((1, B, H), lambda i: (i, 0, 0)),
        ),
        out_shape=(
            jax.ShapeDtypeStruct((NC_ENC, B, SC_ENC), f32),
            jax.ShapeDtypeStruct((NC_ENC, 1, B), f32),
            jax.ShapeDtypeStruct((NC_ENC, 1, B), f32),
            jax.ShapeDtypeStruct((NC_ENC, B, H), f32),
        ),
        compiler_params=pltpu.CompilerParams(
            dimension_semantics=("parallel",),
            vmem_limit_bytes=56 * 1024 * 1024),
        name="enc_attention",
    )(encoder_outputs, q, qb)

    attn3, context, combined = pl.pallas_call(
        _combine_body,
        out_shape=(
            jax.ShapeDtypeStruct((NC_ENC, B, SC_ENC), f32),
            jax.ShapeDtypeStruct((B, H), f32),
            jax.ShapeDtypeStruct((B, H), f32),
        ),
        name="attn_combine",
    )(energies, m_parts, s_parts, ctx_parts, h1, W_comb, bcomb)
    attn_weights = attn3.transpose(1, 0, 2).reshape(B, S_ENC)

    maskf3 = maskf.reshape(B, NC_SEQ, SC_SEQ).transpose(1, 0, 2)
    seq_e3 = pl.pallas_call(
        _seqe_body,
        grid=(NC_SEQ,),
        in_specs=[
            pl.BlockSpec((B, SC_SEQ, ENC2), lambda i: (0, i, 0)),
            pl.BlockSpec((H, ENC2), lambda i: (0, 0)),
            pl.BlockSpec((B, H), lambda i: (0, 0)),
            pl.BlockSpec((1, B, SC_SEQ), lambda i: (i, 0, 0)),
        ],
        out_specs=pl.BlockSpec((1, B, SC_SEQ), lambda i: (i, 0, 0)),
        out_shape=jax.ShapeDtypeStruct((NC_SEQ, B, SC_SEQ), f32),
        compiler_params=pltpu.CompilerParams(
            dimension_semantics=("parallel",),
            vmem_limit_bytes=56 * 1024 * 1024),
        name="seq_energies",
    )(seqs_encoding, W_seqkey, h1, maskf3)
    seq_e = seq_e3.transpose(1, 0, 2).reshape(B, S_SEQ)

    copy3 = pl.pallas_call(
        _scatter_body,
        grid=(NB_SC,),
        in_specs=[
            pl.BlockSpec((BC_SC, S_SEQ), lambda i: (i, 0)),
            pl.BlockSpec((BC_SC, S_SEQ), lambda i: (i, 0)),
        ],
        out_specs=pl.BlockSpec((BC_SC, A_DIM, R_DIM), lambda i: (i, 0, 0)),
        out_shape=jax.ShapeDtypeStruct((B, A_DIM, R_DIM), f32),
        compiler_params=pltpu.CompilerParams(
            dimension_semantics=("parallel",),
            vmem_limit_bytes=56 * 1024 * 1024),
        name="scatter_max_densify",
    )(seq_e, idx32)
    copy_flat = copy3.reshape(B, A_DIM * R_DIM)

    out_main = pl.pallas_call(
        _logits_body,
        grid=(NVC,),
        in_specs=[
            pl.BlockSpec((B, H), lambda i: (0, 0)),
            pl.BlockSpec((VC, H), lambda i: (i, 0)),
            pl.BlockSpec((1, VC), lambda i: (0, i)),
            pl.BlockSpec((B, VC), lambda i: (0, i)),
        ],
        out_specs=pl.BlockSpec((B, VC), lambda i: (0, i)),
        out_shape=jax.ShapeDtypeStruct((B, V), f32),
        compiler_params=pltpu.CompilerParams(
            dimension_semantics=("parallel",),
            vmem_limit_bytes=56 * 1024 * 1024),
        name="logits_copy_add",
    )(combined, W_out, bout, copy_flat)

    copy_oov = copy_flat[:, V:V + NUM_OOV]
    out = pl.pallas_call(
        _lsm_body,
        grid=(2,),
        in_specs=[
            pl.BlockSpec((B // 2, V), lambda i: (i, 0)),
            pl.BlockSpec((B // 2, NUM_OOV), lambda i: (i, 0)),
        ],
        out_specs=pl.BlockSpec((B // 2, V + NUM_OOV), lambda i: (i, 0)),
        out_shape=jax.ShapeDtypeStruct((B, V + NUM_OOV), f32),
        compiler_params=pltpu.CompilerParams(
            dimension_semantics=("parallel",),
            vmem_limit_bytes=56 * 1024 * 1024),
        name="extended_log_softmax",
    )(out_main, copy_oov)

    return (out, context, h1[None], c1[None], attn_weights)
